# baseline (speedup 1.0000x reference)
import functools
import jax
import jax.numpy as jnp
from jax import lax
from jax.experimental import pallas as pl
from jax.experimental.pallas import tpu as pltpu
from jax.experimental.pallas import tpu_sc as plsc

NC = 2
NS = 16
NW = NC * NS
GROUP = 128
BLK = 4
NSTREAM = 2
ROWS_BLK = BLK * GROUP
HI_SHIFT = 6
LO_MASK = 63


@functools.partial(jax.jit, static_argnums=(4, 5))
def _lookup(table, thsc, tlsc, idx3d, n_rows, d):
  n_table = table.shape[0]
  rows_per_w = (n_rows * GROUP) // NW
  nb = rows_per_w // ROWS_BLK
  assert rows_per_w % ROWS_BLK == 0 and nb % 2 == 0 and nb >= 4
  hd = d // 2
  pad_row = thsc.shape[0] // d - 1

  mesh = plsc.VectorSubcoreMesh(core_axis_name="c", subcore_axis_name="s")

  @functools.partial(
      pl.kernel,
      out_type=jax.ShapeDtypeStruct((n_rows * GROUP, d), jnp.float32),
      mesh=mesh,
      compiler_params=pltpu.CompilerParams(
          use_tc_tiling_on_sc=False, needs_layout_passes=False),
      scratch_types=[
          pltpu.VMEM((thsc.shape[0],), jnp.float32),
          pltpu.VMEM((tlsc.shape[0],), jnp.float32),
          pltpu.VMEM((2, NSTREAM, GROUP), jnp.int32),
          pltpu.VMEM((2, BLK - NSTREAM, GROUP), jnp.int32),
          pltpu.VMEM((2, NSTREAM * GROUP, d), jnp.float32),
          pltpu.VMEM((2, (BLK - NSTREAM) * GROUP, d), jnp.float32),
          pltpu.VMEM_SHARED((n_table, d), jnp.float32),
          pltpu.SemaphoreType.DMA,
          pltpu.SemaphoreType.DMA,
          pltpu.SemaphoreType.DMA,
          pltpu.SemaphoreType.DMA,
          pltpu.SemaphoreType.DMA,
          pltpu.SemaphoreType.DMA,
      ],
  )
  def k(table_hbm, thsc_hbm, tlsc_hbm, idx_hbm, out_hbm,
        th_v, tl_v, idx_vs, idx_vf, rows_vs, rows_vf, table_sh,
        isem0, isem1, gsem0, gsem1, ssem0, ssem1):
    isem = (isem0, isem1)
    gsem = (gsem0, gsem1)
    ssem = (ssem0, ssem1)
    wid = lax.axis_index("s") * NC + lax.axis_index("c")
    base = wid * rows_per_w
    gbase = base // GROUP

    @pl.when(lax.axis_index("s") == 0)
    def _():
      pltpu.sync_copy(table_hbm, table_sh)

    pltpu.sync_copy(thsc_hbm, th_v)
    pltpu.sync_copy(tlsc_hbm, tl_v)
    plsc.subcore_barrier()

    iota1 = lax.iota(jnp.int32, 16)
    iota2 = iota1 * 2

    def idx_copies(b, p):
      return [
          pltpu.make_async_copy(
              idx_hbm.at[pl.ds(gbase + b * BLK, NSTREAM)],
              idx_vs.at[p], isem[p]),
          pltpu.make_async_copy(
              idx_hbm.at[pl.ds(gbase + b * BLK + NSTREAM, BLK - NSTREAM)],
              idx_vf.at[p], isem[p]),
      ]

    def idx_start(b, p):
      for c in idx_copies(b, p):
        c.start()

    def idx_wait(b, p):
      for c in idx_copies(b, p):
        c.wait()

    def gather_copies(b, p):
      del b
      return [
          pltpu.make_async_copy(
              table_sh.at[idx_vs.at[p, j]],
              rows_vs.at[p, pl.ds(j * GROUP, GROUP)], gsem[p])
          for j in range(NSTREAM)
      ]

    def store_copies(b, p):
      return [
          pltpu.make_async_copy(
              rows_vs.at[p],
              out_hbm.at[pl.ds(base + b * ROWS_BLK, NSTREAM * GROUP)],
              ssem[p]),
          pltpu.make_async_copy(
              rows_vf.at[p],
              out_hbm.at[pl.ds(base + b * ROWS_BLK + NSTREAM * GROUP,
                               (BLK - NSTREAM) * GROUP)],
              ssem[p]),
      ]

    def fill_half(p, j):
      @plsc.parallel_loop(0, GROUP, 16)
      def _(r0):
        posv = idx_vf[p, j - NSTREAM, pl.ds(r0, 16)]
        hiv = jnp.where(posv == 0, pad_row,
                        lax.shift_right_logical(posv, HI_SHIFT))
        ohv = hiv * d
        olv = (posv & LO_MASK) * d

        @plsc.parallel_loop(0, 16, unroll=8)
        def _(t):
          tv = jnp.full((16,), t, jnp.int32)
          oh = jnp.take_along_axis(ohv, tv, axis=0) + iota1
          ol = jnp.take_along_axis(olv, tv, axis=0) + iota1
          sh0 = plsc.load_gather(th_v, [oh])
          sh1 = plsc.load_gather(th_v, [oh + 16])
          ch0 = plsc.load_gather(th_v, [oh + hd])
          ch1 = plsc.load_gather(th_v, [oh + hd + 16])
          sl0 = plsc.load_gather(tl_v, [ol])
          sl1 = plsc.load_gather(tl_v, [ol + 16])
          cl0 = plsc.load_gather(tl_v, [ol + hd])
          cl1 = plsc.load_gather(tl_v, [ol + hd + 16])
          s0 = sh0 * cl0 + ch0 * sl0
          s1 = sh1 * cl1 + ch1 * sl1
          c0 = ch0 * cl0 - sh0 * sl0
          c1 = ch1 * cl1 - sh1 * sl1
          rowv = jnp.full((16,), (j - NSTREAM) * GROUP + r0 + t, jnp.int32)
          plsc.store_scatter(rows_vf.at[p], [rowv, iota2], s0)
          plsc.store_scatter(rows_vf.at[p], [rowv, iota2 + 1], c0)
          plsc.store_scatter(rows_vf.at[p], [rowv, iota2 + hd], s1)
          plsc.store_scatter(rows_vf.at[p], [rowv, iota2 + hd + 1], c1)

    def fill(p):
      for j in range(NSTREAM, BLK):
        fill_half(p, j)

    def block_iter(b, p, fire_next, wait_store_next, idx_next2):
      q = 1 - p
      if fire_next:
        if wait_store_next:
          for c in store_copies(b - 1, q):
            c.wait()
        idx_wait(b + 1, q)
        for c in gather_copies(b + 1, q):
          c.start()
      fill(p)
      for c in gather_copies(b, p):
        c.wait()
      for c in store_copies(b, p):
        c.start()
      if idx_next2:
        idx_start(b + 2, p)

    idx_start(0, 0)
    idx_start(1, 1)
    idx_wait(0, 0)
    for c in gather_copies(0, 0):
      c.start()

    block_iter(0, 0, True, False, True)
    block_iter(1, 1, True, True, True)

    @pl.loop(1, nb // 2 - 1)
    def _(i):
      b = i * 2
      block_iter(b, 0, True, True, True)
      block_iter(b + 1, 1, True, True, True)

    block_iter(nb - 2, 0, True, True, False)
    block_iter(nb - 1, 1, False, False, False)

    for c in store_copies(nb - 2, 0):
      c.wait()
    for c in store_copies(nb - 1, 1):
      c.wait()

  return k(table, thsc, tlsc, idx3d)


def _deinterleave(t):
  return jnp.concatenate([t[:, 0::2], t[:, 1::2]], axis=1)


def kernel(src_pos, table):
  b, h = src_pos.shape
  n, d = table.shape
  n_rows = (b * h) // GROUP
  unit = jnp.tile(jnp.asarray([0.0, 1.0], table.dtype), d // 2)
  th = table[:: (1 << HI_SHIFT)].at[0].set(unit)
  th = jnp.concatenate([th, jnp.zeros((1, d), table.dtype)], axis=0)
  tl = table[: (1 << HI_SHIFT)].at[0].set(unit)
  thsc = _deinterleave(th).reshape(-1)
  tlsc = _deinterleave(tl).reshape(-1)
  idx3d = src_pos.reshape(n_rows, GROUP)
  out = _lookup(table, thsc, tlsc, idx3d, n_rows, d)
  return out.reshape(b, h, d)


# speedup vs baseline: 1.1209x; 1.1209x over previous
import functools
import jax
import jax.numpy as jnp
from jax import lax
from jax.experimental import pallas as pl
from jax.experimental.pallas import tpu as pltpu
from jax.experimental.pallas import tpu_sc as plsc

NC = 2
NS = 16
NW = NC * NS
GROUP = 128
BLK = 4


@functools.partial(jax.jit, static_argnums=(2, 3))
def _gather(table, idx2d, n_rows, d):
  n_table = table.shape[0]
  rows_per_w = n_rows // NW
  nb = rows_per_w // BLK
  assert rows_per_w % BLK == 0 and nb % 2 == 0 and nb >= 4

  mesh = plsc.VectorSubcoreMesh(core_axis_name="c", subcore_axis_name="s")

  @functools.partial(
      pl.kernel,
      out_type=jax.ShapeDtypeStruct((n_rows, GROUP, d), jnp.float32),
      mesh=mesh,
      compiler_params=pltpu.CompilerParams(use_tc_tiling_on_sc=False),
      scratch_types=[
          pltpu.VMEM((2, BLK, GROUP), jnp.int32),
          pltpu.VMEM((2, BLK, GROUP, d), jnp.float32),
          pltpu.VMEM_SHARED((n_table, d), jnp.float32),
          pltpu.SemaphoreType.DMA,
          pltpu.SemaphoreType.DMA,
          pltpu.SemaphoreType.DMA,
          pltpu.SemaphoreType.DMA,
          pltpu.SemaphoreType.DMA,
          pltpu.SemaphoreType.DMA,
      ],
  )
  def k(table_hbm, idx_hbm, out_hbm, idx_v, rows_v, table_sh,
        isem0, isem1, gsem0, gsem1, ssem0, ssem1):
    isem = (isem0, isem1)
    gsem = (gsem0, gsem1)
    ssem = (ssem0, ssem1)
    wid = lax.axis_index("s") * NC + lax.axis_index("c")
    base = wid * rows_per_w

    @pl.when(lax.axis_index("s") == 0)
    def _():
      pltpu.sync_copy(table_hbm, table_sh)

    plsc.subcore_barrier()

    def idx_copy(b, p):
      return pltpu.make_async_copy(
          idx_hbm.at[pl.ds(base + b * BLK, BLK)], idx_v.at[p], isem[p])

    def gather_copies(b, p):
      del b
      return [
          pltpu.make_async_copy(
              table_sh.at[idx_v.at[p, j]], rows_v.at[p, j], gsem[p])
          for j in range(BLK)
      ]

    def store_copy(b, p):
      return pltpu.make_async_copy(
          rows_v.at[p], out_hbm.at[pl.ds(base + b * BLK, BLK)], ssem[p])

    def block_iter(b, p, fire_next, wait_store_next, idx_next2):
      q = 1 - p
      if fire_next:
        if wait_store_next:
          store_copy(b - 1, q).wait()
        idx_copy(b + 1, q).wait()
        for c in gather_copies(b + 1, q):
          c.start()
      for c in gather_copies(b, p):
        c.wait()
      store_copy(b, p).start()
      if idx_next2:
        idx_copy(b + 2, p).start()

    idx_copy(0, 0).start()
    idx_copy(1, 1).start()
    idx_copy(0, 0).wait()
    for c in gather_copies(0, 0):
      c.start()

    block_iter(0, 0, True, False, True)
    block_iter(1, 1, True, True, True)

    @pl.loop(1, nb // 2 - 1)
    def _(i):
      b = i * 2
      block_iter(b, 0, True, True, True)
      block_iter(b + 1, 1, True, True, True)

    block_iter(nb - 2, 0, True, True, False)
    block_iter(nb - 1, 1, False, False, False)

    store_copy(nb - 2, 0).wait()
    store_copy(nb - 1, 1).wait()

  return k(table, idx2d)


def kernel(src_pos, table):
  b, h = src_pos.shape
  d = table.shape[1]
  n_rows = (b * h) // GROUP
  idx2d = src_pos.reshape(n_rows, GROUP)
  out = _gather(table, idx2d, n_rows, d)
  return out.reshape(b, h, d)
